# baseline (speedup 1.0000x reference)
import jax
import jax.numpy as jnp
from jax import lax
from jax.experimental import pallas as pl
from jax.experimental.pallas import tpu as pltpu
from jax.experimental.pallas import tpu_sc as plsc

import functools


def _make_sc_gather(B, V, D, max_idx):
    info = plsc.get_sparse_core_info()
    NC, NS, L = info.num_cores, info.num_subcores, info.num_lanes
    NW = NC * NS
    assert B % NW == 0 and D % L == 0
    b_per_w = B // NW
    CHUNK = 32
    n_chunks = b_per_w // CHUNK

    n_loaders = max(d for d in range(1, NS + 1)
                    if V % d == 0 and (V // d) % 8 == 0)
    rows_per_loader = V // n_loaders

    mesh = plsc.VectorSubcoreMesh(core_axis_name="c", subcore_axis_name="s")

    @functools.partial(
        pl.kernel,
        mesh=mesh,
        out_type=jax.ShapeDtypeStruct((B, D), jnp.float32),
        scratch_types=[
            pltpu.VMEM((b_per_w,), jnp.float32),
            pltpu.VMEM((b_per_w,), jnp.int32),
            pltpu.VMEM((b_per_w, D), jnp.float32),
            pltpu.VMEM_SHARED((V, D), jnp.float32),
            pltpu.SemaphoreType.DMA,
            pltpu.SemaphoreType.DMA,
            pltpu.SemaphoreType.DMA,
            pltpu.SemaphoreType.DMA,
        ],
    )
    def gather_kernel(pe_hbm, vals_hbm, out_hbm, vals_v, idx_v, rows_v,
                      table_s, sem_t, sem_a, sem_b, sem_w):
        sid = lax.axis_index("s")
        wid = sid * NC + lax.axis_index("c")
        base = wid * b_per_w

        lbase = sid * rows_per_loader

        @pl.when(sid < n_loaders)
        def _start_table():
            pltpu.async_copy(
                pe_hbm.at[pl.ds(lbase, rows_per_loader)],
                table_s.at[pl.ds(lbase, rows_per_loader)],
                sem_t)

        pltpu.sync_copy(vals_hbm.at[pl.ds(base, b_per_w)], vals_v)

        def idx_body(i, carry):
            off = pl.multiple_of(i * L, L)
            v = vals_v[pl.ds(off, L)]
            idx = (v * 1000.0).astype(jnp.int32)
            idx_v[pl.ds(off, L)] = jnp.minimum(jnp.maximum(idx, 0), max_idx)
            return carry

        lax.fori_loop(0, b_per_w // L, idx_body, 0)

        @pl.when(sid < n_loaders)
        def _wait_table():
            pltpu.make_async_copy(
                pe_hbm.at[pl.ds(lbase, rows_per_loader)],
                table_s.at[pl.ds(lbase, rows_per_loader)],
                sem_t).wait()

        plsc.subcore_barrier()

        sems = (sem_a, sem_b)

        def gather_chunk(j):
            return pltpu.async_copy(
                table_s.at[idx_v.at[pl.ds(j * CHUNK, CHUNK)]],
                rows_v.at[pl.ds(j * CHUNK, CHUNK)],
                sems[j % 2],
            )

        g = gather_chunk(0)
        writes = []
        for j in range(n_chunks):
            g_next = gather_chunk(j + 1) if j + 1 < n_chunks else None
            g.wait()
            writes.append(pltpu.async_copy(
                rows_v.at[pl.ds(j * CHUNK, CHUNK)],
                out_hbm.at[pl.ds(base + j * CHUNK, CHUNK)],
                sem_w,
            ))
            g = g_next
        for w in writes:
            w.wait()

    return gather_kernel


def kernel(x, pe, dim_idx):
    vals = lax.dynamic_slice(
        x, (jnp.zeros((), jnp.int32), jnp.asarray(dim_idx, jnp.int32)),
        (x.shape[0], 1)).reshape(x.shape[0])
    B = x.shape[0]
    V, D = pe.shape
    fn = _make_sc_gather(B, V, D, V - 1)
    return fn(pe, vals)


# speedup vs baseline: 1.0121x; 1.0121x over previous
import jax
import jax.numpy as jnp
from jax import lax
from jax.experimental import pallas as pl
from jax.experimental.pallas import tpu as pltpu
from jax.experimental.pallas import tpu_sc as plsc

import functools


def _make_sc_gather(B, V, D, max_idx):
    info = plsc.get_sparse_core_info()
    NC, NS, L = info.num_cores, info.num_subcores, info.num_lanes
    NW = NC * NS
    assert B % NW == 0 and D % L == 0
    b_per_w = B // NW
    CHUNK = 64
    n_chunks = b_per_w // CHUNK

    n_loaders = max(d for d in range(1, NS + 1)
                    if V % d == 0 and (V // d) % 8 == 0)
    rows_per_loader = V // n_loaders

    mesh = plsc.VectorSubcoreMesh(core_axis_name="c", subcore_axis_name="s")

    @functools.partial(
        pl.kernel,
        mesh=mesh,
        out_type=jax.ShapeDtypeStruct((B, D), jnp.float32),
        scratch_types=[
            pltpu.VMEM((b_per_w,), jnp.float32),
            pltpu.VMEM((b_per_w,), jnp.int32),
            pltpu.VMEM((b_per_w, D), jnp.float32),
            pltpu.VMEM_SHARED((V, D), jnp.float32),
            pltpu.SemaphoreType.DMA,
            pltpu.SemaphoreType.DMA,
            pltpu.SemaphoreType.DMA,
            pltpu.SemaphoreType.DMA,
        ],
    )
    def gather_kernel(pe_hbm, vals_hbm, out_hbm, vals_v, idx_v, rows_v,
                      table_s, sem_t, sem_a, sem_b, sem_w):
        sid = lax.axis_index("s")
        wid = sid * NC + lax.axis_index("c")
        base = wid * b_per_w

        lbase = sid * rows_per_loader

        @pl.when(sid < n_loaders)
        def _start_table():
            pltpu.async_copy(
                pe_hbm.at[pl.ds(lbase, rows_per_loader)],
                table_s.at[pl.ds(lbase, rows_per_loader)],
                sem_t)

        pltpu.sync_copy(vals_hbm.at[pl.ds(base, b_per_w)], vals_v)

        def idx_body(i, carry):
            off = pl.multiple_of(i * L, L)
            v = vals_v[pl.ds(off, L)]
            idx = (v * 1000.0).astype(jnp.int32)
            idx_v[pl.ds(off, L)] = jnp.minimum(jnp.maximum(idx, 0), max_idx)
            return carry

        lax.fori_loop(0, b_per_w // L, idx_body, 0)

        @pl.when(sid < n_loaders)
        def _wait_table():
            pltpu.make_async_copy(
                pe_hbm.at[pl.ds(lbase, rows_per_loader)],
                table_s.at[pl.ds(lbase, rows_per_loader)],
                sem_t).wait()

        plsc.subcore_barrier()

        sems = (sem_a, sem_b)

        def gather_chunk(j):
            return pltpu.async_copy(
                table_s.at[idx_v.at[pl.ds(j * CHUNK, CHUNK)]],
                rows_v.at[pl.ds(j * CHUNK, CHUNK)],
                sems[j % 2],
            )

        g = gather_chunk(0)
        writes = []
        for j in range(n_chunks):
            g_next = gather_chunk(j + 1) if j + 1 < n_chunks else None
            g.wait()
            writes.append(pltpu.async_copy(
                rows_v.at[pl.ds(j * CHUNK, CHUNK)],
                out_hbm.at[pl.ds(base + j * CHUNK, CHUNK)],
                sem_w,
            ))
            g = g_next
        for w in writes:
            w.wait()

    return gather_kernel


def kernel(x, pe, dim_idx):
    vals = lax.dynamic_slice(
        x, (jnp.zeros((), jnp.int32), jnp.asarray(dim_idx, jnp.int32)),
        (x.shape[0], 1)).reshape(x.shape[0])
    B = x.shape[0]
    V, D = pe.shape
    fn = _make_sc_gather(B, V, D, V - 1)
    return fn(pe, vals)
